# baseline (speedup 1.0000x reference)
import functools

import jax
import jax.numpy as jnp
from jax import lax
from jax.experimental import pallas as pl
from jax.experimental.pallas import tpu as pltpu
from jax.experimental.pallas import tpu_sc as plsc

_VOCAB = 1000
_D_EMBED = 128
_D_HIDDEN = 2048
_BATCH = 16384

_NC = 2
_NS = 16
_NW = _NC * _NS
_B_PER_W = _BATCH // _NW
_CHUNK = 128
_N_CHUNKS = _B_PER_W // _CHUNK


def _gather_body(a_hbm, b_hbm, ea_hbm, eb_hbm, xa_hbm, xb_hbm,
                 idx_v, rows_v, sem):
    wid = lax.axis_index("s") * _NC + lax.axis_index("c")
    base = wid * _B_PER_W
    for table, out in ((ea_hbm, xa_hbm), (eb_hbm, xb_hbm)):
        idx_hbm = a_hbm if table is ea_hbm else b_hbm
        for j in range(_N_CHUNKS):
            off = base + j * _CHUNK
            pltpu.sync_copy(idx_hbm.at[pl.ds(off, _CHUNK)], idx_v)
            pltpu.async_copy(table.at[idx_v], rows_v, sem).wait()
            pltpu.sync_copy(rows_v, out.at[pl.ds(off, _CHUNK)])


@functools.cache
def _sc_gather():
    return pl.kernel(
        _gather_body,
        out_type=(
            jax.ShapeDtypeStruct((_BATCH, _D_EMBED), jnp.float32),
            jax.ShapeDtypeStruct((_BATCH, _D_EMBED), jnp.float32),
        ),
        mesh=plsc.VectorSubcoreMesh(core_axis_name="c", subcore_axis_name="s"),
        scratch_types=(
            pltpu.VMEM((_CHUNK,), jnp.int32),
            pltpu.VMEM((_CHUNK, _D_EMBED), jnp.float32),
            pltpu.SemaphoreType.DMA,
        ),
    )


_BT = 512


def _mlp_body(xa_ref, xb_ref, wi_ref, wo_ref, out_ref):
    x = xa_ref[...] + xb_ref[...]
    h = jnp.maximum(
        jnp.dot(x, wi_ref[...], preferred_element_type=jnp.float32), 0.0)
    out_ref[...] = jnp.dot(h, wo_ref[...],
                           preferred_element_type=jnp.float32)


def _mlp(xa, xb, w_in_t, w_out_t):
    return pl.pallas_call(
        _mlp_body,
        grid=(_BATCH // _BT,),
        in_specs=[
            pl.BlockSpec((_BT, _D_EMBED), lambda i: (i, 0)),
            pl.BlockSpec((_BT, _D_EMBED), lambda i: (i, 0)),
            pl.BlockSpec((_D_EMBED, _D_HIDDEN), lambda i: (0, 0)),
            pl.BlockSpec((_D_HIDDEN, _VOCAB), lambda i: (0, 0)),
        ],
        out_specs=pl.BlockSpec((_BT, _VOCAB), lambda i: (i, 0)),
        out_shape=jax.ShapeDtypeStruct((_BATCH, _VOCAB), jnp.float32),
    )(xa, xb, w_in_t, w_out_t)


def kernel(a, b, E_a, E_b, W_in, W_out):
    a = a.astype(jnp.int32)
    b = b.astype(jnp.int32)
    xa, xb = _sc_gather()(a, b, E_a, E_b)
    return _mlp(xa, xb, W_in.T, W_out.T)


# speedup vs baseline: 1.0521x; 1.0521x over previous
import functools

import jax
import jax.numpy as jnp
from jax import lax
from jax.experimental import pallas as pl
from jax.experimental.pallas import tpu as pltpu
from jax.experimental.pallas import tpu_sc as plsc

_VOCAB = 1000
_D_EMBED = 128
_D_HIDDEN = 2048
_BATCH = 16384

_NC = 2
_NS = 16
_NW = _NC * _NS
_B_PER_W = _BATCH // _NW
_CHUNK = 128
_N_CHUNKS = _B_PER_W // _CHUNK


def _gather_body(a_hbm, b_hbm, ea_hbm, eb_hbm, xa_hbm, xb_hbm,
                 idx_v, rows_v, sem):
    wid = lax.axis_index("s") * _NC + lax.axis_index("c")
    base = wid * _B_PER_W
    for table, out in ((ea_hbm, xa_hbm), (eb_hbm, xb_hbm)):
        idx_hbm = a_hbm if table is ea_hbm else b_hbm
        for j in range(_N_CHUNKS):
            off = base + j * _CHUNK
            pltpu.sync_copy(idx_hbm.at[pl.ds(off, _CHUNK)], idx_v)
            pltpu.async_copy(table.at[idx_v], rows_v, sem).wait()
            pltpu.sync_copy(rows_v, out.at[pl.ds(off, _CHUNK)])


@functools.cache
def _sc_gather():
    return pl.kernel(
        _gather_body,
        out_type=(
            jax.ShapeDtypeStruct((_BATCH, _D_EMBED), jnp.float32),
            jax.ShapeDtypeStruct((_BATCH, _D_EMBED), jnp.float32),
        ),
        mesh=plsc.VectorSubcoreMesh(core_axis_name="c", subcore_axis_name="s"),
        scratch_types=(
            pltpu.VMEM((_CHUNK,), jnp.int32),
            pltpu.VMEM((_CHUNK, _D_EMBED), jnp.float32),
            pltpu.SemaphoreType.DMA,
        ),
    )


_BT = 512


def _mlp_body(xa_ref, xb_ref, wi_ref, wo_ref, out_ref):
    x = (xa_ref[...] + xb_ref[...]).astype(jnp.bfloat16)
    h = jnp.maximum(
        jnp.dot(x, wi_ref[...], preferred_element_type=jnp.float32), 0.0)
    out_ref[...] = jnp.dot(h.astype(jnp.bfloat16), wo_ref[...],
                           preferred_element_type=jnp.float32)


def _mlp(xa, xb, w_in_t, w_out_t):
    return pl.pallas_call(
        _mlp_body,
        grid=(_BATCH // _BT,),
        in_specs=[
            pl.BlockSpec((_BT, _D_EMBED), lambda i: (i, 0)),
            pl.BlockSpec((_BT, _D_EMBED), lambda i: (i, 0)),
            pl.BlockSpec((_D_EMBED, _D_HIDDEN), lambda i: (0, 0)),
            pl.BlockSpec((_D_HIDDEN, _VOCAB), lambda i: (0, 0)),
        ],
        out_specs=pl.BlockSpec((_BT, _VOCAB), lambda i: (i, 0)),
        out_shape=jax.ShapeDtypeStruct((_BATCH, _VOCAB), jnp.float32),
    )(xa, xb, w_in_t, w_out_t)


def kernel(a, b, E_a, E_b, W_in, W_out):
    a = a.astype(jnp.int32)
    b = b.astype(jnp.int32)
    xa, xb = _sc_gather()(a, b, E_a, E_b)
    return _mlp(xa, xb, W_in.T.astype(jnp.bfloat16),
                W_out.T.astype(jnp.bfloat16))
